# baseline (speedup 1.0000x reference)
import functools

import jax
import jax.numpy as jnp
from jax.experimental import pallas as pl
from jax.experimental.pallas import tpu as pltpu

B, N, M = 4, 4096, 4096
TN = 512
NI = N // TN


def _chamfer_kernel(x1_ref, x2t_ref, d1_ref, d2_ref, loss_ref):
    b = pl.program_id(0)
    i = pl.program_id(1)

    x1 = x1_ref[0]
    x2t = x2t_ref[0]

    inner = jax.lax.dot_general(
        x1, x2t, (((1,), (0,)), ((), ())),
        preferred_element_type=jnp.float32)
    sq1 = jnp.sum(x1 * x1, axis=1)
    sq2 = jnp.sum(x2t * x2t, axis=0)

    d = sq1[:, None] + sq2[None, :] - 2.0 * inner
    d = jnp.maximum(d, 0.0)

    d1 = jnp.min(d, axis=1)
    d1_ref[b, pl.ds(i * TN, TN)] = d1

    part2 = jnp.min(d, axis=0)

    @pl.when(i == 0)
    def _init2():
        d2_ref[b, :] = part2

    @pl.when(i != 0)
    def _acc2():
        d2_ref[b, :] = jnp.minimum(d2_ref[b, :], part2)

    @pl.when(jnp.logical_and(b == 0, i == 0))
    def _init_loss():
        loss_ref[0, 0] = 0.0

    loss_ref[0, 0] += jnp.sum(d1) * (1.0 / (B * N))

    @pl.when(i == NI - 1)
    def _acc_loss2():
        loss_ref[0, 0] += jnp.sum(d2_ref[b, :]) * (1.0 / (B * M))


@jax.jit
def kernel(input1, input2):
    x2t = jnp.transpose(input2, (0, 2, 1))

    grid = (B, NI)
    dist1, dist2, loss = pl.pallas_call(
        _chamfer_kernel,
        grid=grid,
        in_specs=[
            pl.BlockSpec((1, TN, 3), lambda b, i: (b, i, 0)),
            pl.BlockSpec((1, 3, M), lambda b, i: (b, 0, 0)),
        ],
        out_specs=[
            pl.BlockSpec((B, N), lambda b, i: (0, 0)),
            pl.BlockSpec((B, M), lambda b, i: (0, 0)),
            pl.BlockSpec(memory_space=pltpu.SMEM),
        ],
        out_shape=[
            jax.ShapeDtypeStruct((B, N), jnp.float32),
            jax.ShapeDtypeStruct((B, M), jnp.float32),
            jax.ShapeDtypeStruct((1, 1), jnp.float32),
        ],
    )(input1, x2t)
    return (loss[0, 0], dist1, dist2)


# speedup vs baseline: 1.2153x; 1.2153x over previous
import jax
import jax.numpy as jnp
from jax.experimental import pallas as pl
from jax.experimental.pallas import tpu as pltpu

B, N, M = 4, 4096, 4096
TN = 512
NI = N // TN


def _chamfer_kernel(x1_ref, x2t_ref, d1_ref, d2_ref, loss_ref):
    b = pl.program_id(0)
    i = pl.program_id(1)

    x1 = x1_ref[0]
    x2t = x2t_ref[0]

    inner = jax.lax.dot_general(
        x1, x2t, (((1,), (0,)), ((), ())),
        preferred_element_type=jnp.float32)
    sq1 = jnp.sum(x1 * x1, axis=1)
    sq2 = 0.25 * jnp.sum(x2t * x2t, axis=0)

    d = (sq1[:, None] + sq2[None, :]) + inner

    d1 = jnp.maximum(jnp.min(d, axis=1), 0.0)
    d1_ref[b, pl.ds(i * TN, TN)] = d1

    part2 = jnp.min(d, axis=0)

    @pl.when(i == 0)
    def _init2():
        d2_ref[b, :] = part2

    @pl.when(i != 0)
    def _acc2():
        d2_ref[b, :] = jnp.minimum(d2_ref[b, :], part2)

    @pl.when(jnp.logical_and(b == 0, i == 0))
    def _init_loss():
        loss_ref[0, 0] = 0.0

    loss_ref[0, 0] += jnp.sum(d1) * (1.0 / (B * N))

    @pl.when(i == NI - 1)
    def _acc_loss2():
        d2f = jnp.maximum(d2_ref[b, :], 0.0)
        d2_ref[b, :] = d2f
        loss_ref[0, 0] += jnp.sum(d2f) * (1.0 / (B * M))


@jax.jit
def kernel(input1, input2):
    x2t = -2.0 * jnp.transpose(input2, (0, 2, 1))

    grid = (B, NI)
    dist1, dist2, loss = pl.pallas_call(
        _chamfer_kernel,
        grid=grid,
        in_specs=[
            pl.BlockSpec((1, TN, 3), lambda b, i: (b, i, 0)),
            pl.BlockSpec((1, 3, M), lambda b, i: (b, 0, 0)),
        ],
        out_specs=[
            pl.BlockSpec((B, N), lambda b, i: (0, 0)),
            pl.BlockSpec((B, M), lambda b, i: (0, 0)),
            pl.BlockSpec(memory_space=pltpu.SMEM),
        ],
        out_shape=[
            jax.ShapeDtypeStruct((B, N), jnp.float32),
            jax.ShapeDtypeStruct((B, M), jnp.float32),
            jax.ShapeDtypeStruct((1, 1), jnp.float32),
        ],
    )(input1, x2t)
    return (loss[0, 0], dist1, dist2)


# speedup vs baseline: 1.5535x; 1.2783x over previous
import jax
import jax.numpy as jnp
from jax.experimental import pallas as pl
from jax.experimental.pallas import tpu as pltpu

B, N, M = 4, 4096, 4096
TN = 1024
NI = N // TN
LANES = 128


def _chamfer_kernel(x1_ref, x2t_ref, d1_ref, d2_ref, loss_ref, acc1_ref):
    b = pl.program_id(0)
    i = pl.program_id(1)

    x1 = x1_ref[0]
    x2t = x2t_ref[0]

    inner = jax.lax.dot_general(
        x1, x2t, (((1,), (0,)), ((), ())),
        preferred_element_type=jnp.float32)
    sq1 = jnp.sum(x1 * x1, axis=1, keepdims=True)
    sq2 = 0.25 * jnp.sum(x2t * x2t, axis=0, keepdims=True)

    d = (sq1 + sq2) + inner

    fold = d[:, 0:LANES]
    for g in range(1, M // LANES):
        fold = jnp.minimum(fold, d[:, g * LANES:(g + 1) * LANES])
    acc1_ref[pl.ds(i * TN, TN), :] = fold

    part2 = jnp.min(d, axis=0)

    @pl.when(i == 0)
    def _init2():
        d2_ref[b, :] = part2

    @pl.when(i != 0)
    def _acc2():
        d2_ref[b, :] = jnp.minimum(d2_ref[b, :], part2)

    @pl.when(jnp.logical_and(b == 0, i == 0))
    def _init_loss():
        loss_ref[0, 0] = 0.0

    @pl.when(i == NI - 1)
    def _finish():
        d1 = jnp.maximum(jnp.min(acc1_ref[:, :], axis=1), 0.0)
        d1_ref[b, :] = d1
        d2f = jnp.maximum(d2_ref[b, :], 0.0)
        d2_ref[b, :] = d2f
        loss_ref[0, 0] += (jnp.sum(d1) * (1.0 / (B * N))
                           + jnp.sum(d2f) * (1.0 / (B * M)))


@jax.jit
def kernel(input1, input2):
    x2t = -2.0 * jnp.transpose(input2, (0, 2, 1))

    grid = (B, NI)
    dist1, dist2, loss = pl.pallas_call(
        _chamfer_kernel,
        grid=grid,
        in_specs=[
            pl.BlockSpec((1, TN, 3), lambda b, i: (b, i, 0)),
            pl.BlockSpec((1, 3, M), lambda b, i: (b, 0, 0)),
        ],
        out_specs=[
            pl.BlockSpec((B, N), lambda b, i: (0, 0)),
            pl.BlockSpec((B, M), lambda b, i: (0, 0)),
            pl.BlockSpec(memory_space=pltpu.SMEM),
        ],
        out_shape=[
            jax.ShapeDtypeStruct((B, N), jnp.float32),
            jax.ShapeDtypeStruct((B, M), jnp.float32),
            jax.ShapeDtypeStruct((1, 1), jnp.float32),
        ],
        scratch_shapes=[pltpu.VMEM((N, LANES), jnp.float32)],
    )(input1, x2t)
    return (loss[0, 0], dist1, dist2)


# speedup vs baseline: 1.6877x; 1.0864x over previous
import jax
import jax.numpy as jnp
from jax.experimental import pallas as pl
from jax.experimental.pallas import tpu as pltpu

B, N, M = 4, 4096, 4096
TN = 1024
NI = N // TN
LANES = 128


def _chamfer_kernel(x1_ref, x2t_ref, d1_ref, d2_ref, loss_ref, acc1_ref):
    b = pl.program_id(0)
    i = pl.program_id(1)

    x1 = x1_ref[0]
    x2t = x2t_ref[0]

    inner = jax.lax.dot_general(
        x1, x2t, (((1,), (0,)), ((), ())),
        preferred_element_type=jnp.float32)
    sq1 = jnp.sum(x1 * x1, axis=1, keepdims=True)
    sq2 = 0.25 * jnp.sum(x2t * x2t, axis=0, keepdims=True)

    d = (sq1 + sq2) + inner

    fold = d[:, 0:LANES]
    for g in range(1, M // LANES):
        fold = jnp.minimum(fold, d[:, g * LANES:(g + 1) * LANES])
    acc1_ref[pl.ds(i * TN, TN), :] = fold

    part2 = jnp.min(d, axis=0)

    @pl.when(i == 0)
    def _init2():
        d2_ref[b, :] = part2

    @pl.when(i != 0)
    def _acc2():
        d2_ref[b, :] = jnp.minimum(d2_ref[b, :], part2)

    @pl.when(jnp.logical_and(b == 0, i == 0))
    def _init_loss():
        loss_ref[0, 0] = 0.0

    @pl.when(i == NI - 1)
    def _finish():
        acc1_t = jnp.transpose(acc1_ref[:, :], (1, 0))
        d1 = jnp.maximum(jnp.min(acc1_t, axis=0), 0.0)
        d1_ref[b, :] = d1
        d2f = jnp.maximum(d2_ref[b, :], 0.0)
        d2_ref[b, :] = d2f
        loss_ref[0, 0] += (jnp.sum(d1) * (1.0 / (B * N))
                           + jnp.sum(d2f) * (1.0 / (B * M)))


@jax.jit
def kernel(input1, input2):
    x2t = -2.0 * jnp.transpose(input2, (0, 2, 1))

    grid = (B, NI)
    dist1, dist2, loss = pl.pallas_call(
        _chamfer_kernel,
        grid=grid,
        in_specs=[
            pl.BlockSpec((1, TN, 3), lambda b, i: (b, i, 0)),
            pl.BlockSpec((1, 3, M), lambda b, i: (b, 0, 0)),
        ],
        out_specs=[
            pl.BlockSpec((B, N), lambda b, i: (0, 0)),
            pl.BlockSpec((B, M), lambda b, i: (0, 0)),
            pl.BlockSpec(memory_space=pltpu.SMEM),
        ],
        out_shape=[
            jax.ShapeDtypeStruct((B, N), jnp.float32),
            jax.ShapeDtypeStruct((B, M), jnp.float32),
            jax.ShapeDtypeStruct((1, 1), jnp.float32),
        ],
        scratch_shapes=[pltpu.VMEM((N, LANES), jnp.float32)],
    )(input1, x2t)
    return (loss[0, 0], dist1, dist2)
